# baseline (speedup 1.0000x reference)
import jax
import jax.numpy as jnp
from jax import lax
from jax.experimental import pallas as pl
from jax.experimental.pallas import tpu as pltpu
from jax.experimental.pallas import tpu_sc as plsc

_D = 64
_GAMMA = 12.0
_B = 16384

_NC = 2
_NS = 16
_NW = _NC * _NS
_BPW = _B // _NW
_CHUNK = 128
_NCHUNK = _BPW // _CHUNK
_LANES = 16
_CPR = _D // _LANES


def _sc_body(h_idx, r_idx, t_idx, ent, rel, out,
             hv_i, rv_i, tv_i, h_rows, r_rows, t_rows, out_v, acc_buf,
             sem_h, sem_r, sem_t):
  wid = lax.axis_index("s") * _NC + lax.axis_index("c")
  base = wid * _BPW

  pltpu.sync_copy(h_idx.at[pl.ds(wid * _NCHUNK, _NCHUNK), :], hv_i)
  pltpu.sync_copy(r_idx.at[pl.ds(wid * _NCHUNK, _NCHUNK), :], rv_i)
  pltpu.sync_copy(t_idx.at[pl.ds(wid * _NCHUNK, _NCHUNK), :], tv_i)

  copies = []
  for k in range(_NCHUNK):
    dst = pl.ds(k * _CHUNK, _CHUNK)
    copies.append(pltpu.async_copy(ent.at[hv_i.at[k]], h_rows.at[dst, :], sem_h))
    copies.append(pltpu.async_copy(rel.at[rv_i.at[k]], r_rows.at[dst, :], sem_r))
    copies.append(pltpu.async_copy(ent.at[tv_i.at[k]], t_rows.at[dst, :], sem_t))
  for c in copies:
    c.wait()

  lane = lax.iota(jnp.int32, _LANES)

  def group(g, carry):
    for j in range(_LANES):
      b = g * _LANES + j
      acc = None
      for c in range(_CPR):
        sl = pl.ds(c * _LANES, _LANES)
        d = jnp.abs(h_rows[b, sl] + r_rows[b, sl] - t_rows[b, sl])
        acc = d if acc is None else acc + d
      acc_buf[j, :] = acc
    tot = jnp.zeros((_LANES,), jnp.float32)
    for c in range(_LANES):
      col = jnp.full((_LANES,), c, jnp.int32)
      tot = tot + plsc.load_gather(acc_buf, [lane, col])
    out_v[pl.ds(g * _LANES, _LANES)] = _GAMMA - tot
    return carry

  lax.fori_loop(0, _BPW // _LANES, group, 0)
  pltpu.sync_copy(out_v, out.at[pl.ds(base, _BPW)])


@jax.jit
def _sc_score(h_idx, r_idx, t_idx, ent, rel):
  mesh = plsc.VectorSubcoreMesh(
      core_axis_name="c", subcore_axis_name="s",
      num_cores=_NC, num_subcores=_NS)
  return pl.kernel(
      _sc_body,
      out_type=jax.ShapeDtypeStruct((_B,), jnp.float32),
      mesh=mesh,
      compiler_params=pltpu.CompilerParams(
          needs_layout_passes=False, use_tc_tiling_on_sc=False),
      scratch_types=[
          pltpu.VMEM((_NCHUNK, _CHUNK), jnp.int32),
          pltpu.VMEM((_NCHUNK, _CHUNK), jnp.int32),
          pltpu.VMEM((_NCHUNK, _CHUNK), jnp.int32),
          pltpu.VMEM((_BPW, _D), jnp.float32),
          pltpu.VMEM((_BPW, _D), jnp.float32),
          pltpu.VMEM((_BPW, _D), jnp.float32),
          pltpu.VMEM((_BPW,), jnp.float32),
          pltpu.VMEM((_LANES, _LANES), jnp.float32),
          pltpu.SemaphoreType.DMA,
          pltpu.SemaphoreType.DMA,
          pltpu.SemaphoreType.DMA,
      ],
  )(h_idx, r_idx, t_idx, ent, rel)


def kernel(sample, entity_embedding, relation_embedding):
  h = sample[:, 0].astype(jnp.int32).reshape(_B // _CHUNK, _CHUNK)
  r = sample[:, 1].astype(jnp.int32).reshape(_B // _CHUNK, _CHUNK)
  t = sample[:, 2].astype(jnp.int32).reshape(_B // _CHUNK, _CHUNK)
  score = _sc_score(h, r, t, entity_embedding, relation_embedding)
  return score.reshape(_B, 1)
